# Initial kernel scaffold; baseline (speedup 1.0000x reference)
import jax
import jax.numpy as jnp
from jax.experimental import pallas as pl


def kernel(input_ids, token_type_ids, word_emb, pos_emb, type_emb, gamma, beta):
    raise NotImplementedError("write your pallas kernel here")


# speedup vs baseline: 1.5289x; 1.5289x over previous
import functools

import jax
import jax.numpy as jnp
from jax import lax
from jax.experimental import pallas as pl
from jax.experimental.pallas import tpu as pltpu
from jax.experimental.pallas import tpu_sc as plsc

HID = 768
B = 4
S = 2048
EPS = 1e-12

N = B * S
NC = 2
NS = 16
NW = NC * NS
ROWS_PER_W = N // NW
CHUNK = 64
NCHUNK = ROWS_PER_W // CHUNK

ROWS_TC = 256
GRID = N // ROWS_TC
S_BLOCKS = S // ROWS_TC


def _gather_body(ids_hbm, table_hbm, out_hbm, idx_v, buf0, buf1, sem0, sem1):
    wid = lax.axis_index("s") * NC + lax.axis_index("c")
    base = wid * ROWS_PER_W
    pltpu.sync_copy(ids_hbm.at[wid], idx_v)
    bufs = (buf0, buf1)
    sems = (sem0, sem1)
    cps = [pltpu.async_copy(table_hbm.at[idx_v.at[0]], bufs[0], sems[0])]
    for c in range(NCHUNK):
        if c + 1 < NCHUNK:
            cps.append(
                pltpu.async_copy(
                    table_hbm.at[idx_v.at[c + 1]],
                    bufs[(c + 1) % 2],
                    sems[(c + 1) % 2],
                )
            )
        cps[c].wait()
        pltpu.sync_copy(bufs[c % 2], out_hbm.at[pl.ds(base + c * CHUNK, CHUNK)])


def _sc_gather(ids_flat, word_emb):
    mesh = plsc.VectorSubcoreMesh(core_axis_name="c", subcore_axis_name="s")
    ids3 = ids_flat.reshape(NW, NCHUNK, CHUNK)
    run = pl.kernel(
        _gather_body,
        mesh=mesh,
        out_type=jax.ShapeDtypeStruct((N, HID), jnp.float32),
        scratch_types=[
            pltpu.VMEM((NCHUNK, CHUNK), jnp.int32),
            pltpu.VMEM((CHUNK, HID), jnp.float32),
            pltpu.VMEM((CHUNK, HID), jnp.float32),
            pltpu.SemaphoreType.DMA,
            pltpu.SemaphoreType.DMA,
        ],
    )
    return run(ids3, word_emb)


def _ln_body(tt_ref, x_ref, pos_ref, type_ref, gamma_ref, beta_ref, out_ref):
    x = x_ref[...] + pos_ref[...]
    tt = tt_ref[0, 0, :].astype(jnp.float32)[:, None]
    t0 = type_ref[0:1, :]
    t1 = type_ref[1:2, :]
    x = x + t0 + tt * (t1 - t0)
    mean = jnp.mean(x, axis=1, keepdims=True)
    xc = x - mean
    var = jnp.mean(xc * xc, axis=1, keepdims=True)
    inv = lax.rsqrt(var + EPS)
    out_ref[...] = xc * inv * gamma_ref[...] + beta_ref[...]


def _tc_layernorm(tt3, gathered, pos_emb, type_emb, gamma2, beta2):
    return pl.pallas_call(
        _ln_body,
        grid=(GRID,),
        in_specs=[
            pl.BlockSpec((1, 1, ROWS_TC), lambda i: (i, 0, 0)),
            pl.BlockSpec((ROWS_TC, HID), lambda i: (i, 0)),
            pl.BlockSpec((ROWS_TC, HID), lambda i: (i % S_BLOCKS, 0)),
            pl.BlockSpec((2, HID), lambda i: (0, 0)),
            pl.BlockSpec((1, HID), lambda i: (0, 0)),
            pl.BlockSpec((1, HID), lambda i: (0, 0)),
        ],
        out_specs=pl.BlockSpec((ROWS_TC, HID), lambda i: (i, 0)),
        out_shape=jax.ShapeDtypeStruct((N, HID), jnp.float32),
    )(tt3, gathered, pos_emb, type_emb, gamma2, beta2)


def kernel(input_ids, token_type_ids, word_emb, pos_emb, type_emb, gamma, beta):
    ids_flat = input_ids.reshape(N).astype(jnp.int32)
    gathered = _sc_gather(ids_flat, word_emb)
    tt3 = token_type_ids.reshape(GRID, 1, ROWS_TC).astype(jnp.int32)
    out = _tc_layernorm(
        tt3,
        gathered,
        pos_emb,
        type_emb,
        gamma.reshape(1, HID),
        beta.reshape(1, HID),
    )
    return out.reshape(B, S, HID)


# speedup vs baseline: 1.9453x; 1.2724x over previous
import functools

import jax
import jax.numpy as jnp
from jax import lax
from jax.experimental import pallas as pl
from jax.experimental.pallas import tpu as pltpu
from jax.experimental.pallas import tpu_sc as plsc

HID = 768
B = 4
S = 2048
EPS = 1e-12

N = B * S
NC = 2
NS = 16
NW = NC * NS
ROWS_PER_W = N // NW
CHUNK = 64
NCHUNK = ROWS_PER_W // CHUNK

ROWS_TC = S
GRID = N // ROWS_TC


def _gather_body(ids_hbm, table_hbm, out_hbm, idx_v, buf0, buf1, sem0, sem1):
    wid = lax.axis_index("s") * NC + lax.axis_index("c")
    base = wid * ROWS_PER_W
    pltpu.sync_copy(ids_hbm.at[wid], idx_v)
    bufs = (buf0, buf1)
    sems = (sem0, sem1)
    cps = [pltpu.async_copy(table_hbm.at[idx_v.at[0]], bufs[0], sems[0])]
    for c in range(NCHUNK):
        if c + 1 < NCHUNK:
            cps.append(
                pltpu.async_copy(
                    table_hbm.at[idx_v.at[c + 1]],
                    bufs[(c + 1) % 2],
                    sems[(c + 1) % 2],
                )
            )
        cps[c].wait()
        pltpu.sync_copy(bufs[c % 2], out_hbm.at[pl.ds(base + c * CHUNK, CHUNK)])


def _sc_gather(ids_flat, word_emb):
    mesh = plsc.VectorSubcoreMesh(core_axis_name="c", subcore_axis_name="s")
    ids3 = ids_flat.reshape(NW, NCHUNK, CHUNK)
    run = pl.kernel(
        _gather_body,
        mesh=mesh,
        out_type=jax.ShapeDtypeStruct((N, HID), jnp.float32),
        scratch_types=[
            pltpu.VMEM((NCHUNK, CHUNK), jnp.int32),
            pltpu.VMEM((CHUNK, HID), jnp.float32),
            pltpu.VMEM((CHUNK, HID), jnp.float32),
            pltpu.SemaphoreType.DMA,
            pltpu.SemaphoreType.DMA,
        ],
    )
    return run(ids3, word_emb)


def _ln_body(tt_ref, x_ref, pos_ref, type_ref, gamma_ref, beta_ref, out_ref):
    x = x_ref[...] + pos_ref[...]
    tt = tt_ref[0, 0, :].astype(jnp.float32)[:, None]
    t0 = type_ref[0:1, :]
    t1 = type_ref[1:2, :]
    x = x + t0 + tt * (t1 - t0)
    mean = jnp.mean(x, axis=1, keepdims=True)
    xc = x - mean
    var = jnp.mean(xc * xc, axis=1, keepdims=True)
    inv = lax.rsqrt(var + EPS)
    out_ref[...] = xc * inv * gamma_ref[...] + beta_ref[...]


def _tc_layernorm(tt3, gathered, pos_emb, type_emb, gamma2, beta2):
    return pl.pallas_call(
        _ln_body,
        grid=(GRID,),
        in_specs=[
            pl.BlockSpec((1, 1, ROWS_TC), lambda i: (i, 0, 0)),
            pl.BlockSpec((ROWS_TC, HID), lambda i: (i, 0)),
            pl.BlockSpec((ROWS_TC, HID), lambda i: (0, 0)),
            pl.BlockSpec((2, HID), lambda i: (0, 0)),
            pl.BlockSpec((1, HID), lambda i: (0, 0)),
            pl.BlockSpec((1, HID), lambda i: (0, 0)),
        ],
        out_specs=pl.BlockSpec((ROWS_TC, HID), lambda i: (i, 0)),
        out_shape=jax.ShapeDtypeStruct((N, HID), jnp.float32),
    )(tt3, gathered, pos_emb, type_emb, gamma2, beta2)


def kernel(input_ids, token_type_ids, word_emb, pos_emb, type_emb, gamma, beta):
    ids_flat = input_ids.reshape(N).astype(jnp.int32)
    gathered = _sc_gather(ids_flat, word_emb)
    tt3 = token_type_ids.reshape(GRID, 1, ROWS_TC).astype(jnp.int32)
    out = _tc_layernorm(
        tt3,
        gathered,
        pos_emb,
        type_emb,
        gamma.reshape(1, HID),
        beta.reshape(1, HID),
    )
    return out.reshape(B, S, HID)
